# baseline (speedup 1.0000x reference)
import functools

import jax
import jax.numpy as jnp
from jax import lax
from jax.experimental import pallas as pl
from jax.experimental.pallas import tpu as pltpu
from jax.experimental.pallas import tpu_sc as plsc

NUM_EMB = 1000000
DIM = 64
BATCH = 4096
SEQ = 50
TOTAL = BATCH * SEQ
NW = 32
PER_W = TOTAL // NW
GRP = 128
G_PER_CHUNK = 4
CHUNK = G_PER_CHUNK * GRP
NCHUNK = PER_W // CHUNK
TBLK = 128
NFULL = NUM_EMB // TBLK
TAIL = NUM_EMB - NFULL * TBLK
T_ITERS = (NFULL + NW - 1) // NW

_mesh = plsc.VectorSubcoreMesh(core_axis_name="c", subcore_axis_name="s")


@functools.partial(
    pl.kernel,
    out_type=jax.ShapeDtypeStruct((NUM_EMB, 2 * DIM), jnp.float32),
    mesh=_mesh,
    compiler_params=pltpu.CompilerParams(needs_layout_passes=False),
    scratch_types=[
        pltpu.VMEM((DIM, TBLK), jnp.float32),
        pltpu.VMEM((DIM, TBLK), jnp.float32),
        pltpu.VMEM((TBLK, 2 * DIM), jnp.float32),
        pltpu.VMEM((TAIL, DIM), jnp.float32),
        pltpu.VMEM((TAIL, DIM), jnp.float32),
    ],
)
def _interleave(gt_hbm, rt_hbm, gtail_hbm, rtail_hbm, scr_hbm,
                bg, br, ob, tgb, trb):
    wid = lax.axis_index("s") * 2 + lax.axis_index("c")
    iota = lax.iota(jnp.int32, 16)

    def do_block(t, carry):
        b = wid + t * NW

        @pl.when(b < NFULL)
        def _():
            c0 = b * TBLK
            pltpu.sync_copy(gt_hbm.at[:, pl.ds(c0, TBLK)], bg)
            pltpu.sync_copy(rt_hbm.at[:, pl.ds(c0, TBLK)], br)

            def chunk16(cc, carry2):
                row_idx = cc * 16 + iota
                for d in range(DIM):
                    plsc.store_scatter(
                        ob, [row_idx, jnp.full((16,), d, jnp.int32)],
                        bg[d, pl.ds(cc * 16, 16)])
                    plsc.store_scatter(
                        ob, [row_idx, jnp.full((16,), DIM + d, jnp.int32)],
                        br[d, pl.ds(cc * 16, 16)])
                return carry2

            lax.fori_loop(0, TBLK // 16, chunk16, 0)
            pltpu.sync_copy(ob, scr_hbm.at[pl.ds(c0, TBLK), :])

        return carry

    lax.fori_loop(0, T_ITERS, do_block, 0)

    @pl.when(wid == 0)
    def _():
        pltpu.sync_copy(gtail_hbm, tgb)
        pltpu.sync_copy(rtail_hbm, trb)
        for i in range(TAIL):
            for k in range(DIM // 16):
                ob[i, pl.ds(k * 16, 16)] = tgb[i, pl.ds(k * 16, 16)]
                ob[i, pl.ds(DIM + k * 16, 16)] = trb[i, pl.ds(k * 16, 16)]
        pltpu.sync_copy(ob.at[pl.ds(0, TAIL), :],
                        scr_hbm.at[pl.ds(NFULL * TBLK, TAIL), :])


@functools.partial(
    pl.kernel,
    out_type=jax.ShapeDtypeStruct((TOTAL, 2 * DIM), jnp.float32),
    mesh=_mesh,
    compiler_params=pltpu.CompilerParams(needs_layout_passes=False),
    scratch_types=[
        pltpu.VMEM((PER_W,), jnp.int32),
        pltpu.VMEM((CHUNK, 2 * DIM), jnp.float32),
        pltpu.SemaphoreType.DMA,
    ],
)
def _gather(xt_hbm, scr_hbm, out_hbm, idx_v, gbuf, sem):
    wid = lax.axis_index("s") * 2 + lax.axis_index("c")
    pltpu.sync_copy(xt_hbm.at[pl.ds(wid * PER_W, PER_W)], idx_v)

    def body(i, carry):
        copies = []
        for j in range(G_PER_CHUNK):
            src = idx_v.at[pl.ds((i * G_PER_CHUNK + j) * GRP, GRP)]
            copies.append(
                pltpu.async_copy(scr_hbm.at[src],
                                 gbuf.at[pl.ds(j * GRP, GRP)], sem))
        for c in copies:
            c.wait()
        base = wid * PER_W + i * CHUNK
        pltpu.sync_copy(gbuf, out_hbm.at[pl.ds(base, CHUNK), :])
        return carry

    lax.fori_loop(0, PER_W // CHUNK, body, 0)
    rem_groups = (PER_W - (PER_W // CHUNK) * CHUNK) // GRP
    if rem_groups:
        copies = []
        for j in range(rem_groups):
            src = idx_v.at[pl.ds(((PER_W // CHUNK) * G_PER_CHUNK + j) * GRP,
                                 GRP)]
            copies.append(
                pltpu.async_copy(scr_hbm.at[src],
                                 gbuf.at[pl.ds(j * GRP, GRP)], sem))
        for c in copies:
            c.wait()
        base = wid * PER_W + (PER_W // CHUNK) * CHUNK
        pltpu.sync_copy(gbuf.at[pl.ds(0, rem_groups * GRP), :],
                        out_hbm.at[pl.ds(base, rem_groups * GRP), :])


def kernel(x, glove_weight, rand_weight):
    xt = x.T.reshape(TOTAL).astype(jnp.int32)
    scr = _interleave(glove_weight.T, rand_weight.T,
                      glove_weight[NFULL * TBLK:], rand_weight[NFULL * TBLK:])
    out = _gather(xt, scr)
    return out.reshape(SEQ, BATCH, 2 * DIM).transpose(1, 0, 2)


# speedup vs baseline: 1.2647x; 1.2647x over previous
import functools

import jax
import jax.numpy as jnp
from jax import lax
from jax.experimental import pallas as pl
from jax.experimental.pallas import tpu as pltpu
from jax.experimental.pallas import tpu_sc as plsc

NUM_EMB = 1000000
DIM = 64
BATCH = 4096
SEQ = 50
TOTAL = BATCH * SEQ
NW = 32
PER_W = TOTAL // NW
GRP = 128
G_PER_CHUNK = 4
CHUNK = G_PER_CHUNK * GRP
NCHUNK = PER_W // CHUNK
TBLK = 128
NFULL = NUM_EMB // TBLK
TAIL = NUM_EMB - NFULL * TBLK
T_ITERS = (NFULL + NW - 1) // NW

_mesh = plsc.VectorSubcoreMesh(core_axis_name="c", subcore_axis_name="s")


@functools.partial(
    pl.kernel,
    out_type=jax.ShapeDtypeStruct((NUM_EMB, 2 * DIM), jnp.float32),
    mesh=_mesh,
    compiler_params=pltpu.CompilerParams(needs_layout_passes=False),
    scratch_types=[
        pltpu.VMEM((2, DIM, TBLK), jnp.float32),
        pltpu.VMEM((2, DIM, TBLK), jnp.float32),
        pltpu.VMEM((2, TBLK, 2 * DIM), jnp.float32),
        pltpu.VMEM((TAIL, DIM), jnp.float32),
        pltpu.VMEM((TAIL, DIM), jnp.float32),
        pltpu.SemaphoreType.DMA,
        pltpu.SemaphoreType.DMA,
        pltpu.SemaphoreType.DMA,
        pltpu.SemaphoreType.DMA,
    ],
)
def _interleave(gt_hbm, rt_hbm, gtail_hbm, rtail_hbm, scr_hbm,
                bg, br, ob, tgb, trb, sem_i0, sem_i1, sem_o0, sem_o1):
    wid = lax.axis_index("s") * 2 + lax.axis_index("c")
    iota = lax.iota(jnp.int32, 16)
    sems_i = (sem_i0, sem_i1)
    sems_o = (sem_o0, sem_o1)

    def fire(t, p):
        b = wid + t * NW

        @pl.when(b < NFULL)
        def _():
            c0 = b * TBLK
            pltpu.async_copy(gt_hbm.at[:, pl.ds(c0, TBLK)], bg.at[p],
                             sems_i[p])
            pltpu.async_copy(rt_hbm.at[:, pl.ds(c0, TBLK)], br.at[p],
                             sems_i[p])

    def process(t, p):
        b = wid + t * NW

        @pl.when(b < NFULL)
        def _():
            c0 = b * TBLK
            pltpu.make_async_copy(gt_hbm.at[:, pl.ds(c0, TBLK)], bg.at[p],
                                  sems_i[p]).wait()
            pltpu.make_async_copy(rt_hbm.at[:, pl.ds(c0, TBLK)], br.at[p],
                                  sems_i[p]).wait()

            @pl.when(t >= 2)
            def _():
                pltpu.make_async_copy(ob.at[p],
                                      scr_hbm.at[pl.ds(c0, TBLK), :],
                                      sems_o[p]).wait()

            def chunk16(cc, carry2):
                row_idx = cc * 16 + iota
                for d in range(DIM):
                    plsc.store_scatter(
                        ob.at[p], [row_idx, jnp.full((16,), d, jnp.int32)],
                        bg[p, d, pl.ds(cc * 16, 16)])
                    plsc.store_scatter(
                        ob.at[p],
                        [row_idx, jnp.full((16,), DIM + d, jnp.int32)],
                        br[p, d, pl.ds(cc * 16, 16)])
                return carry2

            lax.fori_loop(0, TBLK // 16, chunk16, 0)
            pltpu.async_copy(ob.at[p], scr_hbm.at[pl.ds(c0, TBLK), :],
                             sems_o[p])

    fire(0, 0)

    def body2(t2, carry):
        t = t2 * 2
        fire(t + 1, 1)
        process(t, 0)
        fire(t + 2, 0)
        process(t + 1, 1)
        return carry

    lax.fori_loop(0, (T_ITERS + 1) // 2, body2, 0)

    for p in (0, 1):
        pltpu.make_async_copy(ob.at[p], scr_hbm.at[pl.ds(0, TBLK), :],
                              sems_o[p]).wait()

    @pl.when(wid == 0)
    def _():
        pltpu.sync_copy(gtail_hbm, tgb)
        pltpu.sync_copy(rtail_hbm, trb)
        for i in range(TAIL):
            for k in range(DIM // 16):
                ob[0, i, pl.ds(k * 16, 16)] = tgb[i, pl.ds(k * 16, 16)]
                ob[0, i, pl.ds(DIM + k * 16, 16)] = trb[i, pl.ds(k * 16, 16)]
        pltpu.sync_copy(ob.at[0, pl.ds(0, TAIL), :],
                        scr_hbm.at[pl.ds(NFULL * TBLK, TAIL), :])


@functools.partial(
    pl.kernel,
    out_type=jax.ShapeDtypeStruct((TOTAL, 2 * DIM), jnp.float32),
    mesh=_mesh,
    compiler_params=pltpu.CompilerParams(needs_layout_passes=False),
    scratch_types=[
        pltpu.VMEM((PER_W,), jnp.int32),
        pltpu.VMEM((CHUNK, 2 * DIM), jnp.float32),
        pltpu.SemaphoreType.DMA,
    ],
)
def _gather(xt_hbm, scr_hbm, out_hbm, idx_v, gbuf, sem):
    wid = lax.axis_index("s") * 2 + lax.axis_index("c")
    pltpu.sync_copy(xt_hbm.at[pl.ds(wid * PER_W, PER_W)], idx_v)

    def body(i, carry):
        copies = []
        for j in range(G_PER_CHUNK):
            src = idx_v.at[pl.ds((i * G_PER_CHUNK + j) * GRP, GRP)]
            copies.append(
                pltpu.async_copy(scr_hbm.at[src],
                                 gbuf.at[pl.ds(j * GRP, GRP)], sem))
        for c in copies:
            c.wait()
        base = wid * PER_W + i * CHUNK
        pltpu.sync_copy(gbuf, out_hbm.at[pl.ds(base, CHUNK), :])
        return carry

    lax.fori_loop(0, PER_W // CHUNK, body, 0)
    rem_groups = (PER_W - (PER_W // CHUNK) * CHUNK) // GRP
    if rem_groups:
        copies = []
        for j in range(rem_groups):
            src = idx_v.at[pl.ds(((PER_W // CHUNK) * G_PER_CHUNK + j) * GRP,
                                 GRP)]
            copies.append(
                pltpu.async_copy(scr_hbm.at[src],
                                 gbuf.at[pl.ds(j * GRP, GRP)], sem))
        for c in copies:
            c.wait()
        base = wid * PER_W + (PER_W // CHUNK) * CHUNK
        pltpu.sync_copy(gbuf.at[pl.ds(0, rem_groups * GRP), :],
                        out_hbm.at[pl.ds(base, rem_groups * GRP), :])


def kernel(x, glove_weight, rand_weight):
    xt = x.T.reshape(TOTAL).astype(jnp.int32)
    scr = _interleave(glove_weight.T, rand_weight.T,
                      glove_weight[NFULL * TBLK:], rand_weight[NFULL * TBLK:])
    out = _gather(xt, scr)
    return out.reshape(SEQ, BATCH, 2 * DIM).transpose(1, 0, 2)
